# Initial kernel scaffold; baseline (speedup 1.0000x reference)
import jax
import jax.numpy as jnp
from jax.experimental import pallas as pl


def kernel(x, table):
    raise NotImplementedError("write your pallas kernel here")


# speedup vs baseline: 2.3934x; 2.3934x over previous
import functools

import jax
import jax.numpy as jnp
from jax import lax
from jax.experimental import pallas as pl
from jax.experimental.pallas import tpu as pltpu
from jax.experimental.pallas import tpu_sc as plsc

_VOCAB = 128
_D = 32
_NC = 2
_NS = 16
_NW = _NC * _NS
_L = 16
_C = 1280


@functools.lru_cache(maxsize=None)
def _make_kernel(B: int):
  per_w = B // _NW
  nch = per_w // _C
  assert per_w % _C == 0 and nch % 2 == 0

  mesh = plsc.VectorSubcoreMesh(core_axis_name="c", subcore_axis_name="s")

  @functools.partial(
      pl.kernel,
      out_type=jax.ShapeDtypeStruct((B * _D,), jnp.float32),
      mesh=mesh,
      compiler_params=pltpu.CompilerParams(needs_layout_passes=False),
      scratch_types=[
          pltpu.VMEM((_VOCAB * _D,), jnp.float32),
          pltpu.VMEM((_C,), jnp.int32),
          pltpu.VMEM((_C,), jnp.int32),
          pltpu.VMEM((_C * _D,), jnp.float32),
          pltpu.VMEM((_C * _D,), jnp.float32),
          pltpu.SemaphoreType.DMA,
          pltpu.SemaphoreType.DMA,
          pltpu.SemaphoreType.DMA,
          pltpu.SemaphoreType.DMA,
      ],
  )
  def emb(x_hbm, table_hbm, out_hbm,
          table_v, iv0, iv1, ov0, ov1, si0, si1, so0, so1):
    wid = lax.axis_index("s") * _NC + lax.axis_index("c")
    w_base = wid * per_w
    ivs = (iv0, iv1)
    ovs = (ov0, ov1)
    sis = (si0, si1)
    sos = (so0, so1)

    pltpu.sync_copy(table_hbm, table_v)
    for b in range(2):
      pltpu.async_copy(x_hbm.at[pl.ds(w_base + b * _C, _C)], ivs[b], sis[b])

    def outer(gi, carry):
      for b in range(2):
        g = gi * 2 + b
        iv, ov, si, so = ivs[b], ovs[b], sis[b], sos[b]
        base = w_base + g * _C

        pltpu.make_async_copy(x_hbm.at[pl.ds(w_base, _C)], iv, si).wait()

        @pl.when(gi > 0)
        def _():
          pltpu.make_async_copy(
              ov, out_hbm.at[pl.ds(w_base * _D, _C * _D)], so).wait()

        def igroup(i, c2):
          rows = iv[pl.ds(i * _L, _L)]
          rows32 = rows * _D
          pos32 = (lax.broadcasted_iota(jnp.int32, (_L,), 0) + i * _L) * _D
          for d in range(_D):
            vals = plsc.load_gather(table_v, [rows32 + d])
            plsc.store_scatter(ov, [pos32 + d], vals)
          return c2

        lax.fori_loop(0, _C // _L, igroup, 0, unroll=False)

        nxt = jnp.minimum(g + 2, nch - 1)
        pltpu.async_copy(x_hbm.at[pl.ds(w_base + nxt * _C, _C)], iv, si)
        pltpu.async_copy(ov, out_hbm.at[pl.ds(base * _D, _C * _D)], so)
      return carry

    lax.fori_loop(0, nch // 2, outer, 0, unroll=False)

    for b in range(2):
      pltpu.make_async_copy(x_hbm.at[pl.ds(w_base, _C)], ivs[b], sis[b]).wait()
      pltpu.make_async_copy(
          ovs[b], out_hbm.at[pl.ds(w_base * _D, _C * _D)], sos[b]).wait()

  return emb


def kernel(x, table):
  xf = x.reshape(-1).astype(jnp.int32)
  out = _make_kernel(xf.shape[0])(xf, table.reshape(-1))
  return out.reshape(*x.shape, _D)


# speedup vs baseline: 5.2654x; 2.2000x over previous
import functools

import jax
import jax.numpy as jnp
from jax import lax
from jax.experimental import pallas as pl
from jax.experimental.pallas import tpu as pltpu
from jax.experimental.pallas import tpu_sc as plsc

_VOCAB = 128
_D = 32
_NC = 2
_NS = 16
_NW = _NC * _NS
_L = 16
_C = 1280


@functools.lru_cache(maxsize=None)
def _make_kernel(B: int):
  per_w = B // _NW
  nch = per_w // _C
  assert per_w % _C == 0 and nch % 2 == 0

  mesh = plsc.VectorSubcoreMesh(core_axis_name="c", subcore_axis_name="s")

  @functools.partial(
      pl.kernel,
      out_type=jax.ShapeDtypeStruct((B * _D,), jnp.float32),
      mesh=mesh,
      compiler_params=pltpu.CompilerParams(needs_layout_passes=False),
      scratch_types=[
          pltpu.VMEM((_VOCAB * _D,), jnp.float32),
          pltpu.VMEM((_C,), jnp.int32),
          pltpu.VMEM((_C,), jnp.int32),
          pltpu.VMEM((_C * _D,), jnp.float32),
          pltpu.VMEM((_C * _D,), jnp.float32),
          pltpu.SemaphoreType.DMA,
          pltpu.SemaphoreType.DMA,
          pltpu.SemaphoreType.DMA,
          pltpu.SemaphoreType.DMA,
      ],
  )
  def emb(x_hbm, table_hbm, out_hbm,
          table_v, iv0, iv1, ov0, ov1, si0, si1, so0, so1):
    wid = lax.axis_index("s") * _NC + lax.axis_index("c")
    w_base = wid * per_w
    ivs = (iv0, iv1)
    ovs = (ov0, ov1)
    sis = (si0, si1)
    sos = (so0, so1)

    pltpu.sync_copy(table_hbm, table_v)
    for b in range(2):
      pltpu.async_copy(x_hbm.at[pl.ds(w_base + b * _C, _C)], ivs[b], sis[b])

    def outer(gi, carry):
      for b in range(2):
        g = gi * 2 + b
        iv, ov, si, so = ivs[b], ovs[b], sis[b], sos[b]
        base = w_base + g * _C

        pltpu.make_async_copy(x_hbm.at[pl.ds(w_base, _C)], iv, si).wait()

        @pl.when(gi > 0)
        def _():
          pltpu.make_async_copy(
              ov, out_hbm.at[pl.ds(w_base * _D, _C * _D)], so).wait()

        def igroup(i, c2):
          rows = iv[pl.ds(i * _L, _L)]
          for u in range(_L):
            tb = rows[u] * _D
            ob = (i * _L + u) * _D
            ov[pl.ds(ob, _L)] = table_v[pl.ds(tb, _L)]
            ov[pl.ds(ob + _L, _L)] = table_v[pl.ds(tb + _L, _L)]
          return c2

        lax.fori_loop(0, _C // _L, igroup, 0, unroll=False)

        nxt = jnp.minimum(g + 2, nch - 1)
        pltpu.async_copy(x_hbm.at[pl.ds(w_base + nxt * _C, _C)], iv, si)
        pltpu.async_copy(ov, out_hbm.at[pl.ds(base * _D, _C * _D)], so)
      return carry

    lax.fori_loop(0, nch // 2, outer, 0, unroll=False)

    for b in range(2):
      pltpu.make_async_copy(x_hbm.at[pl.ds(w_base, _C)], ivs[b], sis[b]).wait()
      pltpu.make_async_copy(
          ovs[b], out_hbm.at[pl.ds(w_base * _D, _C * _D)], sos[b]).wait()

  return emb


def kernel(x, table):
  xf = x.reshape(-1).astype(jnp.int32)
  out = _make_kernel(xf.shape[0])(xf, table.reshape(-1))
  return out.reshape(*x.shape, _D)
